# baseline (speedup 1.0000x reference)
import numpy as np
import jax
import jax.numpy as jnp
from jax import lax
from jax.experimental import pallas as pl
from jax.experimental.pallas import tpu as pltpu
from jax.experimental.pallas import tpu_sc as plsc

HW = 16384
N = 2048
BOUND = np.float32(0.01 * 2048)
WLAST = np.float32(0.01 * 2048 - 21.0)
INV_BOUND = np.float32(1.0 / (0.01 * 2048))
INF = np.float32(np.inf)
L = 16
NW = 32
QPW = HW // NW
NCH = N // L
UF = 8
NITF = NCH // UF
UC = 4
CPAD = 80
RCAP = ((N + 63) // 64 + 1) * L
MARGIN = 0x400000
GSTEP = np.float32(2.0 / 127.0)


def _nsub(p):
    return lax.shift_right_logical(p + (L - 1), 4)


SUB = N + CPAD


def _dtm_body(xx_hbm, xy_hbm, out_hbm, px_ref, py_ref, clx_ref, cly_ref,
              sbx_ref, sby_ref,
              c0, c1, c2, c3, c4, c5, c6, c7,
              c8, c9, c10, c11, c12, c13, c14, c15, out_ref):
    cand = (c0, c1, c2, c3, c4, c5, c6, c7,
            c8, c9, c10, c11, c12, c13, c14, c15)
    wid = lax.axis_index("s") * 2 + lax.axis_index("c")
    qbase = wid * QPW
    pltpu.sync_copy(xx_hbm, px_ref)
    pltpu.sync_copy(xy_hbm, py_ref)

    iota = lax.iota(jnp.int32, L)
    inf_v = jnp.full((L,), INF, jnp.float32)
    infbits_v = lax.bitcast_convert_type(inf_v, jnp.int32)

    def merge3(blo, bhi, csort):
        r = jnp.flip(csort)
        l1 = jnp.minimum(bhi, r)
        r2 = jnp.flip(jnp.sort(l1))
        l2 = jnp.minimum(blo, r2)
        h2 = jnp.maximum(blo, r2)
        return jnp.sort(l2), jnp.sort(h2)

    def newton_sqrt(v):
        bits = lax.bitcast_convert_type(v, jnp.int32)
        y = lax.bitcast_convert_type(
            lax.shift_right_arithmetic(bits, 1) + 0x1FBD1DF5, jnp.float32)
        for _ in range(3):
            y = 0.5 * (y + v / y)
        return y

    def splat_q(qi):
        col = (4 * wid + lax.shift_right_logical(qi, 7)).astype(jnp.float32)
        row = (qi & 127).astype(jnp.float32)
        qx = jnp.full((L,), col * GSTEP - 1.0, jnp.float32)
        qy = jnp.full((L,), 1.0 - row * GSTEP, jnp.float32)
        return qx, qy

    def run_pass(xref, yref, base, upass, nit, qs, ts):
        (qxa, qya), (qxb, qyb) = qs
        t21ia, t21ib = ts

        def chunks_append(it, ptrs):
            pxs = [xref[pl.ds(base + (it * upass + u) * L, L)]
                   for u in range(upass)]
            pys = [yref[pl.ds(base + (it * upass + u) * L, L)]
                   for u in range(upass)]
            dxa = [pxs[u] - qxa for u in range(upass)]
            dya = [pys[u] - qya for u in range(upass)]
            dxb = [pxs[u] - qxb for u in range(upass)]
            dyb = [pys[u] - qyb for u in range(upass)]
            da = [dxa[u] * dxa[u] + dya[u] * dya[u] for u in range(upass)]
            db = [dxb[u] * dxb[u] + dyb[u] * dyb[u] for u in range(upass)]
            dba = [lax.bitcast_convert_type(da[u], jnp.int32)
                   for u in range(upass)]
            dbb = [lax.bitcast_convert_type(db[u], jnp.int32)
                   for u in range(upass)]
            ma = [dba[u] < t21ia for u in range(upass)]
            mb = [dbb[u] < t21ib for u in range(upass)]
            pa = [plsc.cumsum(ma[u].astype(jnp.int32)) for u in range(upass)]
            pb = [plsc.cumsum(mb[u].astype(jnp.int32)) for u in range(upass)]
            out = []
            for u in range(upass):
                plsc.store_scatter(cand[u], [pa[u] + ptrs[u] - 1],
                                   da[u], mask=ma[u])
                out.append(ptrs[u]
                           + plsc.all_reduce_population_count(ma[u]))
            for u in range(upass):
                plsc.store_scatter(cand[8 + u], [pb[u] + ptrs[upass + u] - 1],
                                   db[u], mask=mb[u])
                out.append(ptrs[upass + u]
                           + plsc.all_reduce_population_count(mb[u]))
            return tuple(out)

        zeros = (jnp.zeros((L,), jnp.int32),) * (2 * upass)
        ptrs = lax.fori_loop(0, nit, chunks_append, zeros)

        if upass == UC:
            def m16x2(a, b):
                rb = jnp.flip(b)
                lo = jnp.minimum(a, rb)
                hi = jnp.maximum(a, rb)
                return jnp.sort(lo), jnp.sort(hi)

            def m32keep(A, B):
                zlo = jnp.minimum(A[0], jnp.flip(B[1]))
                zhi = jnp.minimum(A[1], jnp.flip(B[0]))
                p = jnp.minimum(zlo, zhi)
                q = jnp.maximum(zlo, zhi)
                return jnp.sort(p), jnp.sort(q)

            def fast_drain():
                cs = []
                for u in range(2 * upass):
                    c = cand[u if u < upass else 4 + u][pl.ds(0, L)]
                    cs.append(jnp.sort(jnp.where(iota < ptrs[u], c, INF)))
                A = m16x2(cs[0], cs[1])
                B = m16x2(cs[2], cs[3])
                xa, ya = m32keep(A, B)
                A = m16x2(cs[4], cs[5])
                B = m16x2(cs[6], cs[7])
                xb, yb = m32keep(A, B)
                return xa, ya, xb, yb

            mx = jnp.maximum(jnp.maximum(ptrs[0], ptrs[1]),
                             jnp.maximum(ptrs[2], ptrs[3]))
            mx = jnp.maximum(mx, jnp.maximum(ptrs[4], ptrs[5]))
            mx = jnp.maximum(mx, jnp.maximum(ptrs[6], ptrs[7]))
            return lax.cond(mx[0] <= L, fast_drain,
                            lambda: _slow_drain(ptrs, upass))

        return _slow_drain(ptrs, upass)

    def _slow_drain(ptrs, upass):
        bloa = bhia = blob = bhib = inf_v
        for u in range(upass):
            p_a = ptrs[u][0]
            p_b = ptrs[upass + u][0]
            nsub = lax.max(_nsub(p_a), _nsub(p_b))

            def sub(i, b, u=u, p_a=p_a, p_b=p_b):
                xa, ya, xb, yb = b
                ca = cand[u][pl.ds(i * L, L)]
                cb = cand[8 + u][pl.ds(i * L, L)]
                ca = jnp.where(iota < (p_a - i * L), ca, INF)
                cb = jnp.where(iota < (p_b - i * L), cb, INF)
                xa, ya = merge3(xa, ya, jnp.sort(ca))
                xb, yb = merge3(xb, yb, jnp.sort(cb))
                return (xa, ya, xb, yb)

            bloa, bhia, blob, bhib = lax.fori_loop(
                0, nsub, sub, (bloa, bhia, blob, bhib))
        return bloa, bhia, blob, bhib

    def pass_full(qs, ts):
        return run_pass(px_ref, py_ref, 0, UF, NITF, qs, ts)

    def extract(blo, bhi):
        s16 = jnp.sum(blo)
        s5 = jnp.sum(jnp.where(iota < 5, bhi, jnp.float32(0.0)))
        return (s16 + s5 + bhi[4] * WLAST) * INV_BOUND

    qs0 = (splat_q(jnp.int32(0)), splat_q(jnp.int32(8)))
    r0 = pass_full(qs0, (infbits_v, infbits_v))
    m21 = jnp.maximum(r0[1][4], r0[3][4])
    rs2_v = jnp.full((L,), 4.0 * m21)
    rs2bits_v = lax.bitcast_convert_type(rs2_v, jnp.int32)
    wh = 2.0 * GSTEP + newton_sqrt(rs2_v)
    wh2 = wh * wh
    cx = jnp.full((L,), (4.0 * wid.astype(jnp.float32) + 1.5) * GSTEP - 1.0,
                  jnp.float32)

    def build(it, ptr):
        for u in range(UC):
            px = px_ref[pl.ds((it * UC + u) * L, L)]
            py = py_ref[pl.ds((it * UC + u) * L, L)]
            t = px - cx
            mask = t * t < wh2
            pc = plsc.cumsum(mask.astype(jnp.int32))
            idx = pc + ptr - 1
            plsc.store_scatter(clx_ref, [idx], px, mask=mask)
            plsc.store_scatter(cly_ref, [idx], py, mask=mask)
            ptr = ptr + plsc.all_reduce_population_count(mask)
        return ptr

    llen_v = lax.fori_loop(0, NCH // UC, build,
                           jnp.zeros((L,), jnp.int32))
    ab_v = llen_v & ~15
    plsc.store_scatter(clx_ref, [ab_v + iota], inf_v,
                       mask=iota >= (llen_v & 15))
    plsc.store_scatter(cly_ref, [ab_v + iota], inf_v,
                       mask=iota >= (llen_v & 15))
    ab = ab_v[0]
    for k2 in range(1, CPAD // L):
        clx_ref[pl.ds(ab + k2 * L, L)] = inf_v
        cly_ref[pl.ds(ab + k2 * L, L)] = inf_v
    llen = llen_v[0]
    nit_c = lax.shift_right_logical(llen + (UC * L - 1), 6)

    rsv = newton_sqrt(rs2_v)
    wy2 = (7.5 * GSTEP + rsv) * (7.5 * GSTEP + rsv)
    nits_v = jnp.zeros((L,), jnp.int32)
    for b in range(8):
        yc = jnp.full((L,), np.float32(1.0) - (16 * b + 7.5) * GSTEP,
                      jnp.float32)

        def build_b(it, ptr, yc=yc, b=b):
            for u in range(UC):
                px = clx_ref[pl.ds((it * UC + u) * L, L)]
                py = cly_ref[pl.ds((it * UC + u) * L, L)]
                t = py - yc
                mask = t * t < wy2
                pc = plsc.cumsum(mask.astype(jnp.int32))
                idx = pc + ptr + (b * SUB - 1)
                plsc.store_scatter(sbx_ref, [idx], px, mask=mask)
                plsc.store_scatter(sby_ref, [idx], py, mask=mask)
                ptr = ptr + plsc.all_reduce_population_count(mask)
            return ptr

        lb_v = lax.fori_loop(0, nit_c, build_b, jnp.zeros((L,), jnp.int32))
        ab2_v = lb_v & ~15
        pad_mask = iota >= (lb_v & 15)
        plsc.store_scatter(sbx_ref, [ab2_v + iota + b * SUB], inf_v,
                           mask=pad_mask)
        plsc.store_scatter(sby_ref, [ab2_v + iota + b * SUB], inf_v,
                           mask=pad_mask)
        ab2 = ab2_v[0]
        for k2 in range(1, CPAD // L):
            sbx_ref[pl.ds(b * SUB + ab2 + k2 * L, L)] = inf_v
            sby_ref[pl.ds(b * SUB + ab2 + k2 * L, L)] = inf_v
        nit_b = lax.shift_right_logical(lb_v + (UC * L - 1), 6)
        nits_v = jnp.where(iota == b, nit_b, nits_v)

    def pass_compact(base, nit, qs, ts):
        return run_pass(sbx_ref, sby_ref, base, UC, nit, qs, ts)

    def group_body(g, carry):
        bin_b = g & 7
        base = bin_b * SUB
        nit_b = jnp.sum(jnp.where(iota == bin_b, nits_v, jnp.int32(0)))

        def pair_body(lp, carry):
            outacc, t0a, t0b = carry
            qi = g * L + lp
            qs = (splat_q(qi), splat_q(qi + 8))
            res = pass_compact(base, nit_b, qs, (t0a, t0b))
            failed = jnp.isinf(res[1][4]) | jnp.isinf(res[3][4])
            res = lax.cond(
                failed,
                lambda: pass_full(qs, (infbits_v, infbits_v)),
                lambda: res)
            bloa, bhia, blob, bhib = res
            va = extract(bloa, bhia)
            vb = extract(blob, bhib)
            outacc = jnp.where(iota == lp, va, outacc)
            outacc = jnp.where(iota == lp + 8, vb, outacc)
            t0a = jnp.minimum(lax.bitcast_convert_type(
                jnp.full((L,), bhia[4]), jnp.int32) + MARGIN, rs2bits_v)
            t0b = jnp.minimum(lax.bitcast_convert_type(
                jnp.full((L,), bhib[4]), jnp.int32) + MARGIN, rs2bits_v)
            return (outacc, t0a, t0b)

        t0a, t0b = carry
        outacc, t0a, t0b = lax.fori_loop(
            0, 8, pair_body, (inf_v, t0a, t0b))
        out_ref[pl.ds(g * L, L)] = newton_sqrt(outacc)
        return (t0a, t0b)

    lax.fori_loop(0, QPW // L, group_body,
                  (jnp.minimum(infbits_v, rs2bits_v),
                   jnp.minimum(infbits_v, rs2bits_v)))
    pltpu.sync_copy(out_ref, out_hbm.at[pl.ds(qbase, QPW)])


_dtm = pl.kernel(
    _dtm_body,
    out_type=jax.ShapeDtypeStruct((HW,), jnp.float32),
    mesh=plsc.VectorSubcoreMesh(core_axis_name="c", subcore_axis_name="s"),
    compiler_params=pltpu.CompilerParams(needs_layout_passes=False),
    scratch_types=[
        pltpu.VMEM((N,), jnp.float32),
        pltpu.VMEM((N,), jnp.float32),
        pltpu.VMEM((N + CPAD,), jnp.float32),
        pltpu.VMEM((N + CPAD,), jnp.float32),
        pltpu.VMEM((8 * SUB,), jnp.float32),
        pltpu.VMEM((8 * SUB,), jnp.float32),
    ] + [pltpu.VMEM((RCAP,), jnp.float32)] * 16
    + [pltpu.VMEM((QPW,), jnp.float32)],
)


def kernel(x):
    return _dtm(x[:, 0], x[:, 1]).reshape(128, 128).T


# speedup vs baseline: 1.1083x; 1.1083x over previous
import numpy as np
import jax
import jax.numpy as jnp
from jax import lax
from jax.experimental import pallas as pl
from jax.experimental.pallas import tpu as pltpu
from jax.experimental.pallas import tpu_sc as plsc

HW = 16384
N = 2048
BOUND = np.float32(0.01 * 2048)
WLAST = np.float32(0.01 * 2048 - 21.0)
INV_BOUND = np.float32(1.0 / (0.01 * 2048))
INF = np.float32(np.inf)
L = 16
NW = 32
QPW = HW // NW
NCH = N // L
UF = 8
NITF = NCH // UF
UC = 4
CPAD = 80
RCAP = ((N + 63) // 64 + 1) * L
MARGIN = 0x800000
GSTEP = np.float32(2.0 / 127.0)


def _nsub(p):
    return lax.shift_right_logical(p + (L - 1), 4)


SUB = N + CPAD


def _dtm_body(xx_hbm, xy_hbm, out_hbm, px_ref, py_ref, clx_ref, cly_ref,
              sbx_ref, sby_ref,
              c0, c1, c2, c3, c4, c5, c6, c7,
              c8, c9, c10, c11, c12, c13, c14, c15, out_ref):
    cand = (c0, c1, c2, c3, c4, c5, c6, c7,
            c8, c9, c10, c11, c12, c13, c14, c15)
    wid = lax.axis_index("s") * 2 + lax.axis_index("c")
    qbase = wid * QPW
    pltpu.sync_copy(xx_hbm, px_ref)
    pltpu.sync_copy(xy_hbm, py_ref)

    iota = lax.iota(jnp.int32, L)
    inf_v = jnp.full((L,), INF, jnp.float32)
    infbits_v = lax.bitcast_convert_type(inf_v, jnp.int32)

    def merge3(blo, bhi, csort):
        r = jnp.flip(csort)
        l1 = jnp.minimum(bhi, r)
        r2 = jnp.flip(jnp.sort(l1))
        l2 = jnp.minimum(blo, r2)
        h2 = jnp.maximum(blo, r2)
        return jnp.sort(l2), jnp.sort(h2)

    def newton_sqrt(v):
        bits = lax.bitcast_convert_type(v, jnp.int32)
        y = lax.bitcast_convert_type(
            lax.shift_right_arithmetic(bits, 1) + 0x1FBD1DF5, jnp.float32)
        for _ in range(3):
            y = 0.5 * (y + v / y)
        return y

    def splat_q(qi):
        col = (4 * wid + lax.shift_right_logical(qi, 7)).astype(jnp.float32)
        row = (qi & 127).astype(jnp.float32)
        qx = jnp.full((L,), col * GSTEP - 1.0, jnp.float32)
        qy = jnp.full((L,), 1.0 - row * GSTEP, jnp.float32)
        return qx, qy

    def run_pass(xref, yref, base, upass, nit, qs, ts):
        (qxa, qya), (qxb, qyb) = qs
        t21ia, t21ib = ts

        def chunks_append(it, ptrs):
            pxs = [xref[pl.ds(base + (it * upass + u) * L, L)]
                   for u in range(upass)]
            pys = [yref[pl.ds(base + (it * upass + u) * L, L)]
                   for u in range(upass)]
            dxa = [pxs[u] - qxa for u in range(upass)]
            dya = [pys[u] - qya for u in range(upass)]
            dxb = [pxs[u] - qxb for u in range(upass)]
            dyb = [pys[u] - qyb for u in range(upass)]
            da = [dxa[u] * dxa[u] + dya[u] * dya[u] for u in range(upass)]
            db = [dxb[u] * dxb[u] + dyb[u] * dyb[u] for u in range(upass)]
            dba = [lax.bitcast_convert_type(da[u], jnp.int32)
                   for u in range(upass)]
            dbb = [lax.bitcast_convert_type(db[u], jnp.int32)
                   for u in range(upass)]
            ma = [dba[u] < t21ia for u in range(upass)]
            mb = [dbb[u] < t21ib for u in range(upass)]
            pa = [plsc.cumsum(ma[u].astype(jnp.int32)) for u in range(upass)]
            pb = [plsc.cumsum(mb[u].astype(jnp.int32)) for u in range(upass)]
            out = []
            for u in range(upass):
                plsc.store_scatter(cand[u], [pa[u] + ptrs[u] - 1],
                                   da[u], mask=ma[u])
                out.append(ptrs[u]
                           + plsc.all_reduce_population_count(ma[u]))
            for u in range(upass):
                plsc.store_scatter(cand[8 + u], [pb[u] + ptrs[upass + u] - 1],
                                   db[u], mask=mb[u])
                out.append(ptrs[upass + u]
                           + plsc.all_reduce_population_count(mb[u]))
            return tuple(out)

        zeros = (jnp.zeros((L,), jnp.int32),) * (2 * upass)
        ptrs = lax.fori_loop(0, nit, chunks_append, zeros)

        if upass == UC:
            def m16x2(a, b):
                rb = jnp.flip(b)
                lo = jnp.minimum(a, rb)
                hi = jnp.maximum(a, rb)
                return jnp.sort(lo), jnp.sort(hi)

            def m32keep(A, B):
                zlo = jnp.minimum(A[0], jnp.flip(B[1]))
                zhi = jnp.minimum(A[1], jnp.flip(B[0]))
                p = jnp.minimum(zlo, zhi)
                q = jnp.maximum(zlo, zhi)
                return jnp.sort(p), jnp.sort(q)

            def fast_drain():
                cs = []
                for u in range(2 * upass):
                    c = cand[u if u < upass else 4 + u][pl.ds(0, L)]
                    cs.append(jnp.sort(jnp.where(iota < ptrs[u], c, INF)))
                A = m16x2(cs[0], cs[1])
                B = m16x2(cs[2], cs[3])
                xa, ya = m32keep(A, B)
                A = m16x2(cs[4], cs[5])
                B = m16x2(cs[6], cs[7])
                xb, yb = m32keep(A, B)
                return xa, ya, xb, yb

            mx = jnp.maximum(jnp.maximum(ptrs[0], ptrs[1]),
                             jnp.maximum(ptrs[2], ptrs[3]))
            mx = jnp.maximum(mx, jnp.maximum(ptrs[4], ptrs[5]))
            mx = jnp.maximum(mx, jnp.maximum(ptrs[6], ptrs[7]))
            return lax.cond(mx[0] <= L, fast_drain,
                            lambda: _slow_drain(ptrs, upass))

        return _slow_drain(ptrs, upass)

    def _slow_drain(ptrs, upass):
        bloa = bhia = blob = bhib = inf_v
        for u in range(upass):
            p_a = ptrs[u][0]
            p_b = ptrs[upass + u][0]
            nsub = lax.max(_nsub(p_a), _nsub(p_b))

            def sub(i, b, u=u, p_a=p_a, p_b=p_b):
                xa, ya, xb, yb = b
                ca = cand[u][pl.ds(i * L, L)]
                cb = cand[8 + u][pl.ds(i * L, L)]
                ca = jnp.where(iota < (p_a - i * L), ca, INF)
                cb = jnp.where(iota < (p_b - i * L), cb, INF)
                xa, ya = merge3(xa, ya, jnp.sort(ca))
                xb, yb = merge3(xb, yb, jnp.sort(cb))
                return (xa, ya, xb, yb)

            bloa, bhia, blob, bhib = lax.fori_loop(
                0, nsub, sub, (bloa, bhia, blob, bhib))
        return bloa, bhia, blob, bhib

    def pass_full(qs, ts):
        return run_pass(px_ref, py_ref, 0, UF, NITF, qs, ts)

    def extract(blo, bhi):
        s16 = jnp.sum(blo)
        s5 = jnp.sum(jnp.where(iota < 5, bhi, jnp.float32(0.0)))
        return (s16 + s5 + bhi[4] * WLAST) * INV_BOUND

    qs0 = (splat_q(jnp.int32(0)), splat_q(jnp.int32(8)))
    r0 = pass_full(qs0, (infbits_v, infbits_v))
    m21 = jnp.maximum(r0[1][4], r0[3][4])
    rs2_v = jnp.full((L,), 4.0 * m21)
    rs2bits_v = lax.bitcast_convert_type(rs2_v, jnp.int32)
    wh = 2.0 * GSTEP + newton_sqrt(rs2_v)
    wh2 = wh * wh
    cx = jnp.full((L,), (4.0 * wid.astype(jnp.float32) + 1.5) * GSTEP - 1.0,
                  jnp.float32)

    def build(it, ptr):
        for u in range(UC):
            px = px_ref[pl.ds((it * UC + u) * L, L)]
            py = py_ref[pl.ds((it * UC + u) * L, L)]
            t = px - cx
            mask = t * t < wh2
            pc = plsc.cumsum(mask.astype(jnp.int32))
            idx = pc + ptr - 1
            plsc.store_scatter(clx_ref, [idx], px, mask=mask)
            plsc.store_scatter(cly_ref, [idx], py, mask=mask)
            ptr = ptr + plsc.all_reduce_population_count(mask)
        return ptr

    llen_v = lax.fori_loop(0, NCH // UC, build,
                           jnp.zeros((L,), jnp.int32))
    ab_v = llen_v & ~15
    plsc.store_scatter(clx_ref, [ab_v + iota], inf_v,
                       mask=iota >= (llen_v & 15))
    plsc.store_scatter(cly_ref, [ab_v + iota], inf_v,
                       mask=iota >= (llen_v & 15))
    ab = ab_v[0]
    for k2 in range(1, CPAD // L):
        clx_ref[pl.ds(ab + k2 * L, L)] = inf_v
        cly_ref[pl.ds(ab + k2 * L, L)] = inf_v
    llen = llen_v[0]
    nit_c = lax.shift_right_logical(llen + (UC * L - 1), 6)

    rsv = newton_sqrt(rs2_v)
    wy2 = (7.5 * GSTEP + rsv) * (7.5 * GSTEP + rsv)
    nits_v = jnp.zeros((L,), jnp.int32)
    for b in range(8):
        yc = jnp.full((L,), np.float32(1.0) - (16 * b + 7.5) * GSTEP,
                      jnp.float32)

        def build_b(it, ptr, yc=yc, b=b):
            for u in range(UC):
                px = clx_ref[pl.ds((it * UC + u) * L, L)]
                py = cly_ref[pl.ds((it * UC + u) * L, L)]
                t = py - yc
                mask = t * t < wy2
                pc = plsc.cumsum(mask.astype(jnp.int32))
                idx = pc + ptr + (b * SUB - 1)
                plsc.store_scatter(sbx_ref, [idx], px, mask=mask)
                plsc.store_scatter(sby_ref, [idx], py, mask=mask)
                ptr = ptr + plsc.all_reduce_population_count(mask)
            return ptr

        lb_v = lax.fori_loop(0, nit_c, build_b, jnp.zeros((L,), jnp.int32))
        ab2_v = lb_v & ~15
        pad_mask = iota >= (lb_v & 15)
        plsc.store_scatter(sbx_ref, [ab2_v + iota + b * SUB], inf_v,
                           mask=pad_mask)
        plsc.store_scatter(sby_ref, [ab2_v + iota + b * SUB], inf_v,
                           mask=pad_mask)
        ab2 = ab2_v[0]
        for k2 in range(1, CPAD // L):
            sbx_ref[pl.ds(b * SUB + ab2 + k2 * L, L)] = inf_v
            sby_ref[pl.ds(b * SUB + ab2 + k2 * L, L)] = inf_v
        nit_b = lax.shift_right_logical(lb_v + (UC * L - 1), 6)
        nits_v = jnp.where(iota == b, nit_b, nits_v)

    def pass_compact(base, nit, qs, ts):
        return run_pass(sbx_ref, sby_ref, base, UC, nit, qs, ts)

    def group_body(g, carry):
        bin_b = g & 7
        base = bin_b * SUB
        nit_b = jnp.sum(jnp.where(iota == bin_b, nits_v, jnp.int32(0)))

        def pair_body(lp, carry):
            outacc, t0a, t0b = carry
            qi = g * L + lp
            qs = (splat_q(qi), splat_q(qi + 8))
            res = pass_compact(base, nit_b, qs, (t0a, t0b))
            failed = jnp.isinf(res[1][4]) | jnp.isinf(res[3][4])
            res = lax.cond(
                failed,
                lambda: pass_full(qs, (infbits_v, infbits_v)),
                lambda: res)
            bloa, bhia, blob, bhib = res
            va = extract(bloa, bhia)
            vb = extract(blob, bhib)
            outacc = jnp.where(iota == lp, va, outacc)
            outacc = jnp.where(iota == lp + 8, vb, outacc)
            t0a = jnp.minimum(lax.bitcast_convert_type(
                jnp.full((L,), bhia[4]), jnp.int32) + MARGIN, rs2bits_v)
            t0b = jnp.minimum(lax.bitcast_convert_type(
                jnp.full((L,), bhib[4]), jnp.int32) + MARGIN, rs2bits_v)
            return (outacc, t0a, t0b)

        t0a, t0b = carry
        outacc, t0a, t0b = lax.fori_loop(
            0, 8, pair_body, (inf_v, t0a, t0b))
        out_ref[pl.ds(g * L, L)] = newton_sqrt(outacc)
        return (t0a, t0b)

    lax.fori_loop(0, QPW // L, group_body,
                  (jnp.minimum(infbits_v, rs2bits_v),
                   jnp.minimum(infbits_v, rs2bits_v)))
    pltpu.sync_copy(out_ref, out_hbm.at[pl.ds(qbase, QPW)])


_dtm = pl.kernel(
    _dtm_body,
    out_type=jax.ShapeDtypeStruct((HW,), jnp.float32),
    mesh=plsc.VectorSubcoreMesh(core_axis_name="c", subcore_axis_name="s"),
    compiler_params=pltpu.CompilerParams(needs_layout_passes=False),
    scratch_types=[
        pltpu.VMEM((N,), jnp.float32),
        pltpu.VMEM((N,), jnp.float32),
        pltpu.VMEM((N + CPAD,), jnp.float32),
        pltpu.VMEM((N + CPAD,), jnp.float32),
        pltpu.VMEM((8 * SUB,), jnp.float32),
        pltpu.VMEM((8 * SUB,), jnp.float32),
    ] + [pltpu.VMEM((RCAP,), jnp.float32)] * 16
    + [pltpu.VMEM((QPW,), jnp.float32)],
)


def kernel(x):
    return _dtm(x[:, 0], x[:, 1]).reshape(128, 128).T
